# baseline (speedup 1.0000x reference)
import math

import jax
import jax.numpy as jnp
from jax import lax
from jax.experimental import pallas as pl
from jax.experimental.pallas import tpu as pltpu

INV_E = 1.0 / math.e
BF = jnp.bfloat16
F32 = jnp.float32


def _fused_kernel(x_ref, w1_ref, b1_ref, w2_ref, b2_ref,
                  wa1_ref, ba1_ref, wa2_ref, ba2_ref,
                  w5_ref, b5_ref, wcat1_ref, g2_ref, wc_ref, bc_ref,
                  a2_ref, out_ref, lg_ref):
    for r in range(x_ref.shape[0]):
        _one_row(x_ref[r], w1_ref, b1_ref, w2_ref, b2_ref,
                 wa1_ref, ba1_ref, wa2_ref, ba2_ref,
                 w5_ref, b5_ref, wcat1_ref, g2_ref, wc_ref, bc_ref,
                 a2_ref, out_ref.at[r], lg_ref.at[r])


def _one_row(x0, w1_ref, b1_ref, w2_ref, b2_ref,
             wa1_ref, ba1_ref, wa2_ref, ba2_ref,
             w5_ref, b5_ref, wcat1_ref, g2_ref, wc_ref, bc_ref,
             a2_ref, out_ref, lg_ref):
    T = x0.shape[0]

    h = jnp.dot(x0.astype(BF), w1_ref[...], preferred_element_type=F32) + b1_ref[...]
    h = jnp.maximum(h, 0.0)
    feat = jnp.dot(h.astype(BF), w2_ref[...], preferred_element_type=F32) + b2_ref[...]
    feat = jnp.maximum(feat, 0.0)
    fb = feat.astype(BF)
    a = jnp.dot(fb, wa1_ref[...], preferred_element_type=F32) + ba1_ref[...]
    a = jnp.maximum(a, 0.0)
    a = jnp.dot(a.astype(BF), wa2_ref[...], preferred_element_type=F32) + ba2_ref[...]
    a = jnp.maximum(a, 0.0)

    at = jnp.transpose(a, (1, 0)).astype(BF)
    zt = jnp.dot(w5_ref[...], at, preferred_element_type=F32)
    ztp = jnp.concatenate([jnp.zeros((5, 4), F32), zt], axis=1)
    lgt = ztp[0:1, 0:T]
    for k in range(1, 5):
        lgt = lgt + ztp[k:k + 1, k:k + T]
    lgt = lgt + b5_ref[...]

    lgcol = jnp.transpose(jnp.broadcast_to(lgt, (8, T)), (1, 0))
    lgc = lgcol[:, 0:1]
    lg_ref[...] = lgc

    inv_norm = lax.rsqrt(jnp.sum(x0 * x0, axis=-1, keepdims=True) + 1e-20)
    xn = (x0 * inv_norm).astype(BF)
    sim = lax.dot_general(xn, xn, (((1,), (1,)), ((), ())),
                          preferred_element_type=F32)
    sim = jnp.where(sim > 0.7, sim, 0.0)
    e1 = jnp.exp(sim)
    r1 = pl.reciprocal(jnp.sum(e1, axis=-1, keepdims=True), approx=True)

    a2b = a2_ref[...]

    sj = pl.reciprocal(1.0 + jnp.exp(-lgt), approx=True)
    si = pl.reciprocal(1.0 + jnp.exp(-lgc), approx=True)
    sb = jnp.broadcast_to(sj, (T, T))
    sbt = jnp.broadcast_to(si, (T, T))
    d = jnp.abs(sbt - sb)
    g = pl.reciprocal(1.0 + jnp.exp(d * 10.0 - 5.0), approx=True)
    e3 = jnp.exp(g)
    r3 = pl.reciprocal(jnp.sum(e3, axis=-1, keepdims=True), approx=True)

    a1b = e1.astype(BF)
    a3b = e3.astype(BF)

    sr = jnp.dot(fb, wcat1_ref[...], preferred_element_type=F32)
    h1 = jnp.maximum(jnp.dot(a1b, sr[:, 0:32].astype(BF),
                             preferred_element_type=F32) * r1 + sr[:, 96:128], 0.0)
    h2 = jnp.maximum(jnp.dot(a2b, sr[:, 32:64].astype(BF),
                             preferred_element_type=F32) + sr[:, 128:160], 0.0)
    h3 = jnp.maximum(jnp.dot(a3b, sr[:, 64:96].astype(BF),
                             preferred_element_type=F32) * r3 + sr[:, 160:192], 0.0)

    sc1 = jnp.dot(h1.astype(BF), g2_ref[0:32, :], preferred_element_type=F32)
    sc2 = jnp.dot(h2.astype(BF), g2_ref[32:64, :], preferred_element_type=F32)
    sc3 = jnp.dot(h3.astype(BF), g2_ref[64:96, :], preferred_element_type=F32)
    o1 = jnp.maximum(jnp.dot(a1b, sc1.astype(BF),
                             preferred_element_type=F32) * r1 + h1, 0.0)
    o2 = jnp.maximum(jnp.dot(a2b, sc2.astype(BF),
                             preferred_element_type=F32) + h2, 0.0)
    o3 = jnp.maximum(jnp.dot(a3b, sc3.astype(BF),
                             preferred_element_type=F32) * r3 + h3, 0.0)

    out = jnp.dot(o1.astype(BF), wc_ref[0:32, :], preferred_element_type=F32)
    out = out + jnp.dot(o2.astype(BF), wc_ref[32:64, :], preferred_element_type=F32)
    out = out + jnp.dot(o3.astype(BF), wc_ref[64:96, :], preferred_element_type=F32)
    out_ref[...] = out + bc_ref[...]


def kernel(inputs, w1, b1, w2, b2, wa1, ba1, wa2, ba2, w5, b5,
           wcat1, wblk2, wc, bc):
    B, T, F = inputs.shape
    n_class = wc.shape[1]

    w1b = w1.astype(BF)
    w2b = w2.astype(BF)
    wa1b = wa1.astype(BF)
    wa2b = wa2.astype(BF)
    wcat1p = jnp.pad(wcat1, ((0, 0), (0, 64))).astype(BF)
    w5b = w5.astype(BF)
    ids = jnp.arange(T, dtype=jnp.int32)
    a2c = jnp.exp(jnp.abs(ids[:, None] - ids[None, :]).astype(F32)
                  * (-INV_E)).astype(BF)
    g2stack = jnp.concatenate([wblk2[0:32, 0:32], wblk2[32:64, 32:64],
                               wblk2[64:96, 64:96]], axis=0).astype(BF)
    wcb = wc.astype(BF)

    out, logits = pl.pallas_call(
        _fused_kernel,
        out_shape=(jax.ShapeDtypeStruct((B, T, n_class), F32),
                   jax.ShapeDtypeStruct((B, T, 1), F32)),
        grid=(B // 8,),
        in_specs=[
            pl.BlockSpec((8, T, F), lambda b: (b, 0, 0)),
            pl.BlockSpec((F, 512), lambda b: (0, 0)),
            pl.BlockSpec((1, 512), lambda b: (0, 0)),
            pl.BlockSpec((512, 128), lambda b: (0, 0)),
            pl.BlockSpec((1, 128), lambda b: (0, 0)),
            pl.BlockSpec((128, 64), lambda b: (0, 0)),
            pl.BlockSpec((1, 64), lambda b: (0, 0)),
            pl.BlockSpec((64, 32), lambda b: (0, 0)),
            pl.BlockSpec((1, 32), lambda b: (0, 0)),
            pl.BlockSpec((5, 32), lambda b: (0, 0)),
            pl.BlockSpec((1, 1), lambda b: (0, 0)),
            pl.BlockSpec((128, 256), lambda b: (0, 0)),
            pl.BlockSpec((96, 32), lambda b: (0, 0)),
            pl.BlockSpec((96, n_class), lambda b: (0, 0)),
            pl.BlockSpec((1, n_class), lambda b: (0, 0)),
            pl.BlockSpec((T, T), lambda b: (0, 0)),
        ],
        out_specs=(pl.BlockSpec((8, T, n_class), lambda b: (b, 0, 0)),
                   pl.BlockSpec((8, T, 1), lambda b: (b, 0, 0))),
        compiler_params=pltpu.CompilerParams(
            dimension_semantics=("parallel",),
            vmem_limit_bytes=64 * 1024 * 1024),
    )(inputs, w1b, b1, w2b, b2, wa1b, ba1, wa2b, ba2, w5b, b5,
      wcat1p, g2stack, wcb, bc, a2c)

    return out, logits


# speedup vs baseline: 1.0525x; 1.0525x over previous
import math

import jax
import jax.numpy as jnp
from jax import lax
from jax.experimental import pallas as pl
from jax.experimental.pallas import tpu as pltpu

INV_E = 1.0 / math.e
BF = jnp.bfloat16
F32 = jnp.float32


def _fused_kernel(x_ref, w1_ref, b1_ref, w2_ref, b2_ref,
                  wa1_ref, ba1_ref, wa2_ref, ba2_ref,
                  w5_ref, b5_ref, wcat1_ref, g2_ref, wc_ref, bc_ref,
                  a2_ref, out_ref, lg_ref):
    for r in range(x_ref.shape[0]):
        _one_row(x_ref[r], w1_ref, b1_ref, w2_ref, b2_ref,
                 wa1_ref, ba1_ref, wa2_ref, ba2_ref,
                 w5_ref, b5_ref, wcat1_ref, g2_ref, wc_ref, bc_ref,
                 a2_ref, out_ref.at[r], lg_ref.at[r])


def _one_row(x0, w1_ref, b1_ref, w2_ref, b2_ref,
             wa1_ref, ba1_ref, wa2_ref, ba2_ref,
             w5_ref, b5_ref, wcat1_ref, g2_ref, wc_ref, bc_ref,
             a2_ref, out_ref, lg_ref):
    T = x0.shape[0]

    h = jnp.dot(x0.astype(BF), w1_ref[...], preferred_element_type=F32) + b1_ref[...]
    h = jnp.maximum(h, 0.0)
    feat = jnp.dot(h.astype(BF), w2_ref[...], preferred_element_type=F32) + b2_ref[...]
    feat = jnp.maximum(feat, 0.0)
    fb = feat.astype(BF)
    a = jnp.dot(fb, wa1_ref[...], preferred_element_type=F32) + ba1_ref[...]
    a = jnp.maximum(a, 0.0)
    a = jnp.dot(a.astype(BF), wa2_ref[...], preferred_element_type=F32) + ba2_ref[...]
    a = jnp.maximum(a, 0.0)

    at = jnp.transpose(a, (1, 0)).astype(BF)
    zt = jnp.dot(w5_ref[...], at, preferred_element_type=F32)
    ztp = jnp.concatenate([jnp.zeros((5, 4), F32), zt], axis=1)
    lgt = ztp[0:1, 0:T]
    for k in range(1, 5):
        lgt = lgt + ztp[k:k + 1, k:k + T]
    lgt = lgt + b5_ref[...]

    lgcol = jnp.transpose(jnp.broadcast_to(lgt, (8, T)), (1, 0))
    lgc = lgcol[:, 0:1]
    lg_ref[...] = lgc

    inv_norm = lax.rsqrt(jnp.sum(x0 * x0, axis=-1, keepdims=True) + 1e-20)
    xn = (x0 * inv_norm).astype(BF)
    sim = lax.dot_general(xn, xn, (((1,), (1,)), ((), ())),
                          preferred_element_type=F32)
    sim = jnp.where(sim > 0.7, sim, 0.0)
    e1 = jnp.exp(sim)
    r1 = pl.reciprocal(jnp.sum(e1, axis=-1, keepdims=True), approx=True)

    a2b = a2_ref[...]

    sj = pl.reciprocal(1.0 + jnp.exp(-lgt), approx=True)
    sb = jnp.broadcast_to(sj, (T, T))
    sbt = jnp.transpose(sb, (1, 0))
    d = jnp.abs(sbt - sb)
    g = pl.reciprocal(1.0 + jnp.exp(d * 10.0 - 5.0), approx=True)
    e3 = jnp.exp(g)
    r3 = pl.reciprocal(jnp.sum(e3, axis=-1, keepdims=True), approx=True)

    a1b = e1.astype(BF)
    a3b = e3.astype(BF)

    sr = jnp.dot(fb, wcat1_ref[...], preferred_element_type=F32)
    h1 = jnp.maximum(jnp.dot(a1b, sr[:, 0:32].astype(BF),
                             preferred_element_type=F32) * r1 + sr[:, 96:128], 0.0)
    h2 = jnp.maximum(jnp.dot(a2b, sr[:, 32:64].astype(BF),
                             preferred_element_type=F32) + sr[:, 128:160], 0.0)
    h3 = jnp.maximum(jnp.dot(a3b, sr[:, 64:96].astype(BF),
                             preferred_element_type=F32) * r3 + sr[:, 160:192], 0.0)

    sc1 = jnp.dot(h1.astype(BF), g2_ref[0:32, :], preferred_element_type=F32)
    sc2 = jnp.dot(h2.astype(BF), g2_ref[32:64, :], preferred_element_type=F32)
    sc3 = jnp.dot(h3.astype(BF), g2_ref[64:96, :], preferred_element_type=F32)
    o1 = jnp.maximum(jnp.dot(a1b, sc1.astype(BF),
                             preferred_element_type=F32) * r1 + h1, 0.0)
    o2 = jnp.maximum(jnp.dot(a2b, sc2.astype(BF),
                             preferred_element_type=F32) + h2, 0.0)
    o3 = jnp.maximum(jnp.dot(a3b, sc3.astype(BF),
                             preferred_element_type=F32) * r3 + h3, 0.0)

    out = jnp.dot(o1.astype(BF), wc_ref[0:32, :], preferred_element_type=F32)
    out = out + jnp.dot(o2.astype(BF), wc_ref[32:64, :], preferred_element_type=F32)
    out = out + jnp.dot(o3.astype(BF), wc_ref[64:96, :], preferred_element_type=F32)
    out_ref[...] = out + bc_ref[...]


def kernel(inputs, w1, b1, w2, b2, wa1, ba1, wa2, ba2, w5, b5,
           wcat1, wblk2, wc, bc):
    B, T, F = inputs.shape
    n_class = wc.shape[1]

    w1b = w1.astype(BF)
    w2b = w2.astype(BF)
    wa1b = wa1.astype(BF)
    wa2b = wa2.astype(BF)
    wcat1p = jnp.pad(wcat1, ((0, 0), (0, 64))).astype(BF)
    w5b = w5.astype(BF)
    ids = jnp.arange(T, dtype=jnp.int32)
    a2c = jnp.exp(jnp.abs(ids[:, None] - ids[None, :]).astype(F32)
                  * (-INV_E)).astype(BF)
    g2stack = jnp.concatenate([wblk2[0:32, 0:32], wblk2[32:64, 32:64],
                               wblk2[64:96, 64:96]], axis=0).astype(BF)
    wcb = wc.astype(BF)

    out, logits = pl.pallas_call(
        _fused_kernel,
        out_shape=(jax.ShapeDtypeStruct((B, T, n_class), F32),
                   jax.ShapeDtypeStruct((B, T, 1), F32)),
        grid=(B // 8,),
        in_specs=[
            pl.BlockSpec((8, T, F), lambda b: (b, 0, 0)),
            pl.BlockSpec((F, 512), lambda b: (0, 0)),
            pl.BlockSpec((1, 512), lambda b: (0, 0)),
            pl.BlockSpec((512, 128), lambda b: (0, 0)),
            pl.BlockSpec((1, 128), lambda b: (0, 0)),
            pl.BlockSpec((128, 64), lambda b: (0, 0)),
            pl.BlockSpec((1, 64), lambda b: (0, 0)),
            pl.BlockSpec((64, 32), lambda b: (0, 0)),
            pl.BlockSpec((1, 32), lambda b: (0, 0)),
            pl.BlockSpec((5, 32), lambda b: (0, 0)),
            pl.BlockSpec((1, 1), lambda b: (0, 0)),
            pl.BlockSpec((128, 256), lambda b: (0, 0)),
            pl.BlockSpec((96, 32), lambda b: (0, 0)),
            pl.BlockSpec((96, n_class), lambda b: (0, 0)),
            pl.BlockSpec((1, n_class), lambda b: (0, 0)),
            pl.BlockSpec((T, T), lambda b: (0, 0)),
        ],
        out_specs=(pl.BlockSpec((8, T, n_class), lambda b: (b, 0, 0)),
                   pl.BlockSpec((8, T, 1), lambda b: (b, 0, 0))),
        compiler_params=pltpu.CompilerParams(
            dimension_semantics=("parallel",),
            vmem_limit_bytes=64 * 1024 * 1024),
    )(inputs, w1b, b1, w2b, b2, wa1b, ba1, wa2b, ba2, w5b, b5,
      wcat1p, g2stack, wcb, bc, a2c)

    return out, logits


# speedup vs baseline: 1.3318x; 1.2654x over previous
import math

import jax
import jax.numpy as jnp
from jax import lax
from jax.experimental import pallas as pl
from jax.experimental.pallas import tpu as pltpu

INV_E = 1.0 / math.e
BF = jnp.bfloat16
F32 = jnp.float32


def _fused_kernel(x_ref, w1_ref, b1_ref, w2_ref, b2_ref,
                  wa1_ref, ba1_ref, wa2_ref, ba2_ref,
                  w5_ref, b5_ref, wcat1_ref, g2_ref, wc_ref, bc_ref,
                  a2_ref, out_ref, lg_ref):
    R, T, F = x_ref.shape
    X = x_ref[...].reshape(R * T, F)
    H = jnp.dot(X.astype(BF), w1_ref[...], preferred_element_type=F32) + b1_ref[...]
    H = jnp.maximum(H, 0.0)
    FEAT = jnp.dot(H.astype(BF), w2_ref[...], preferred_element_type=F32) + b2_ref[...]
    FEAT = jnp.maximum(FEAT, 0.0)
    FB = FEAT.astype(BF)
    A = jnp.dot(FB, wa1_ref[...], preferred_element_type=F32) + ba1_ref[...]
    A = jnp.maximum(A, 0.0)
    A = jnp.dot(A.astype(BF), wa2_ref[...], preferred_element_type=F32) + ba2_ref[...]
    A = jnp.maximum(A, 0.0)
    AT = jnp.transpose(A, (1, 0)).astype(BF)
    ZT = jnp.dot(w5_ref[...], AT, preferred_element_type=F32)
    SR = jnp.dot(FB, wcat1_ref[...], preferred_element_type=F32)

    for r in range(R):
        _one_row(x_ref[r], ZT[:, r * T:(r + 1) * T],
                 SR[r * T:(r + 1) * T, :],
                 b5_ref, g2_ref, wc_ref, bc_ref,
                 a2_ref, out_ref.at[r], lg_ref.at[r])


def _one_row(x0, zt, sr, b5_ref, g2_ref, wc_ref, bc_ref,
             a2_ref, out_ref, lg_ref):
    T = x0.shape[0]

    ztp = jnp.concatenate([jnp.zeros((5, 4), F32), zt], axis=1)
    lgt = ztp[0:1, 0:T]
    for k in range(1, 5):
        lgt = lgt + ztp[k:k + 1, k:k + T]
    lgt = lgt + b5_ref[...]

    lgcol = jnp.transpose(jnp.broadcast_to(lgt, (8, T)), (1, 0))
    lgc = lgcol[:, 0:1]
    lg_ref[...] = lgc

    inv_norm = lax.rsqrt(jnp.sum(x0 * x0, axis=-1, keepdims=True) + 1e-20)
    xn = (x0 * inv_norm).astype(BF)
    sim = lax.dot_general(xn, xn, (((1,), (1,)), ((), ())),
                          preferred_element_type=F32)
    sim = jnp.where(sim > 0.7, sim, 0.0)
    e1 = jnp.exp(sim)
    r1 = pl.reciprocal(jnp.sum(e1, axis=-1, keepdims=True), approx=True)

    a2b = a2_ref[...]

    sj = pl.reciprocal(1.0 + jnp.exp(-lgt), approx=True)
    sb = jnp.broadcast_to(sj, (T, T))
    sbt = jnp.transpose(sb, (1, 0))
    d = jnp.abs(sbt - sb)
    g = pl.reciprocal(1.0 + jnp.exp(d * 10.0 - 5.0), approx=True)
    e3 = jnp.exp(g)
    r3 = pl.reciprocal(jnp.sum(e3, axis=-1, keepdims=True), approx=True)

    a1b = e1.astype(BF)
    a3b = e3.astype(BF)

    h1 = jnp.maximum(jnp.dot(a1b, sr[:, 0:32].astype(BF),
                             preferred_element_type=F32) * r1 + sr[:, 96:128], 0.0)
    h2 = jnp.maximum(jnp.dot(a2b, sr[:, 32:64].astype(BF),
                             preferred_element_type=F32) + sr[:, 128:160], 0.0)
    h3 = jnp.maximum(jnp.dot(a3b, sr[:, 64:96].astype(BF),
                             preferred_element_type=F32) * r3 + sr[:, 160:192], 0.0)

    sc1 = jnp.dot(h1.astype(BF), g2_ref[0:32, :], preferred_element_type=F32)
    sc2 = jnp.dot(h2.astype(BF), g2_ref[32:64, :], preferred_element_type=F32)
    sc3 = jnp.dot(h3.astype(BF), g2_ref[64:96, :], preferred_element_type=F32)
    o1 = jnp.maximum(jnp.dot(a1b, sc1.astype(BF),
                             preferred_element_type=F32) * r1 + h1, 0.0)
    o2 = jnp.maximum(jnp.dot(a2b, sc2.astype(BF),
                             preferred_element_type=F32) + h2, 0.0)
    o3 = jnp.maximum(jnp.dot(a3b, sc3.astype(BF),
                             preferred_element_type=F32) * r3 + h3, 0.0)

    out = jnp.dot(o1.astype(BF), wc_ref[0:32, :], preferred_element_type=F32)
    out = out + jnp.dot(o2.astype(BF), wc_ref[32:64, :], preferred_element_type=F32)
    out = out + jnp.dot(o3.astype(BF), wc_ref[64:96, :], preferred_element_type=F32)
    out_ref[...] = out + bc_ref[...]


def kernel(inputs, w1, b1, w2, b2, wa1, ba1, wa2, ba2, w5, b5,
           wcat1, wblk2, wc, bc):
    B, T, F = inputs.shape
    n_class = wc.shape[1]

    w1b = w1.astype(BF)
    w2b = w2.astype(BF)
    wa1b = wa1.astype(BF)
    wa2b = wa2.astype(BF)
    wcat1p = jnp.pad(wcat1, ((0, 0), (0, 64))).astype(BF)
    w5b = w5.astype(BF)
    ids = jnp.arange(T, dtype=jnp.int32)
    a2c = jnp.exp(jnp.abs(ids[:, None] - ids[None, :]).astype(F32)
                  * (-INV_E)).astype(BF)
    g2stack = jnp.concatenate([wblk2[0:32, 0:32], wblk2[32:64, 32:64],
                               wblk2[64:96, 64:96]], axis=0).astype(BF)
    wcb = wc.astype(BF)

    out, logits = pl.pallas_call(
        _fused_kernel,
        out_shape=(jax.ShapeDtypeStruct((B, T, n_class), F32),
                   jax.ShapeDtypeStruct((B, T, 1), F32)),
        grid=(B // 8,),
        in_specs=[
            pl.BlockSpec((8, T, F), lambda b: (b, 0, 0)),
            pl.BlockSpec((F, 512), lambda b: (0, 0)),
            pl.BlockSpec((1, 512), lambda b: (0, 0)),
            pl.BlockSpec((512, 128), lambda b: (0, 0)),
            pl.BlockSpec((1, 128), lambda b: (0, 0)),
            pl.BlockSpec((128, 64), lambda b: (0, 0)),
            pl.BlockSpec((1, 64), lambda b: (0, 0)),
            pl.BlockSpec((64, 32), lambda b: (0, 0)),
            pl.BlockSpec((1, 32), lambda b: (0, 0)),
            pl.BlockSpec((5, 32), lambda b: (0, 0)),
            pl.BlockSpec((1, 1), lambda b: (0, 0)),
            pl.BlockSpec((128, 256), lambda b: (0, 0)),
            pl.BlockSpec((96, 32), lambda b: (0, 0)),
            pl.BlockSpec((96, n_class), lambda b: (0, 0)),
            pl.BlockSpec((1, n_class), lambda b: (0, 0)),
            pl.BlockSpec((T, T), lambda b: (0, 0)),
        ],
        out_specs=(pl.BlockSpec((8, T, n_class), lambda b: (b, 0, 0)),
                   pl.BlockSpec((8, T, 1), lambda b: (b, 0, 0))),
        compiler_params=pltpu.CompilerParams(
            dimension_semantics=("parallel",),
            vmem_limit_bytes=64 * 1024 * 1024),
    )(inputs, w1b, b1, w2b, b2, wa1b, ba1, wa2b, ba2, w5b, b5,
      wcat1p, g2stack, wcb, bc, a2c)

    return out, logits
